# baseline (speedup 1.0000x reference)
import functools

import jax
import jax.numpy as jnp
from jax import lax
from jax.experimental import pallas as pl
from jax.experimental.pallas import tpu as pltpu
from jax.experimental.pallas import tpu_sc as plsc

DIM = 64
LANES = 16
NUM_CORES = 2
NUM_SUBCORES = 16
NUM_WORKERS = NUM_CORES * NUM_SUBCORES
CHUNK = 128


def _sc_transe(total):
    per_w = total // NUM_WORKERS
    n_chunks = per_w // CHUNK
    groups = CHUNK // LANES

    mesh = plsc.VectorSubcoreMesh(
        core_axis_name="c", subcore_axis_name="s",
        num_cores=NUM_CORES, num_subcores=NUM_SUBCORES)

    @functools.partial(
        pl.kernel,
        out_type=jax.ShapeDtypeStruct((total,), jnp.float32),
        mesh=mesh,
        compiler_params=pltpu.CompilerParams(
            needs_layout_passes=False, use_tc_tiling_on_sc=False),
        scratch_types=[
            pltpu.VMEM((n_chunks, CHUNK), jnp.int32),
            pltpu.VMEM((n_chunks, CHUNK), jnp.int32),
            pltpu.VMEM((n_chunks, CHUNK), jnp.int32),
            pltpu.VMEM((CHUNK, DIM), jnp.float32),
            pltpu.VMEM((CHUNK, DIM), jnp.float32),
            pltpu.VMEM((CHUNK, DIM), jnp.float32),
            pltpu.VMEM((per_w,), jnp.float32),
            pltpu.SemaphoreType.DMA,
        ],
    )
    def k(hidx_hbm, ridx_hbm, tidx_hbm, ent_hbm, rel_hbm, out_hbm,
          hidx_v, ridx_v, tidx_v, h_v, r_v, t_v, out_v, sem):
        wid = lax.axis_index("s") * NUM_CORES + lax.axis_index("c")
        row0 = wid * n_chunks
        pltpu.sync_copy(hidx_hbm.at[pl.ds(row0, n_chunks)], hidx_v)
        pltpu.sync_copy(ridx_hbm.at[pl.ds(row0, n_chunks)], ridx_v)
        pltpu.sync_copy(tidx_hbm.at[pl.ds(row0, n_chunks)], tidx_v)

        lane = jnp.arange(LANES, dtype=jnp.int32)

        for j in range(n_chunks):
            pltpu.async_copy(ent_hbm.at[hidx_v.at[j]], h_v, sem)
            pltpu.async_copy(rel_hbm.at[ridx_v.at[j]], r_v, sem)
            pltpu.async_copy(ent_hbm.at[tidx_v.at[j]], t_v, sem)
            pltpu.make_async_copy(ent_hbm.at[hidx_v.at[j]], h_v, sem).wait()
            pltpu.make_async_copy(rel_hbm.at[ridx_v.at[j]], r_v, sem).wait()
            pltpu.make_async_copy(ent_hbm.at[tidx_v.at[j]], t_v, sem).wait()

            def group_body(g, _, j=j):
                rows = g * LANES + lane

                def d_body(d, acc):
                    col = jnp.full((LANES,), d, dtype=jnp.int32)
                    hv = plsc.load_gather(h_v, [rows, col])
                    rv = plsc.load_gather(r_v, [rows, col])
                    tv = plsc.load_gather(t_v, [rows, col])
                    return acc + jnp.abs(hv + rv - tv)

                acc = lax.fori_loop(
                    0, DIM, d_body, jnp.zeros((LANES,), jnp.float32))
                out_v[pl.ds(j * CHUNK + g * LANES, LANES)] = acc
                return 0

            lax.fori_loop(0, groups, group_body, 0)

        pltpu.sync_copy(out_v, out_hbm.at[pl.ds(wid * per_w, per_w)])

    return k


def kernel(positive_triplets, negative_triplets, entities_emb, relations_emb):
    batch = positive_triplets.shape[0]
    total = 2 * batch
    trip = jnp.concatenate(
        [positive_triplets, negative_triplets], axis=0).astype(jnp.int32)
    n_rows = total // CHUNK
    hidx = trip[:, 0].reshape(n_rows, CHUNK)
    ridx = trip[:, 1].reshape(n_rows, CHUNK)
    tidx = trip[:, 2].reshape(n_rows, CHUNK)

    out = _sc_transe(total)(hidx, ridx, tidx, entities_emb, relations_emb)
    return out[:batch], out[batch:]


# speedup vs baseline: 1.4216x; 1.4216x over previous
import functools

import jax
import jax.numpy as jnp
from jax import lax
from jax.experimental import pallas as pl
from jax.experimental.pallas import tpu as pltpu
from jax.experimental.pallas import tpu_sc as plsc

DIM = 64
LANES = 16
NUM_CORES = 2
NUM_SUBCORES = 16
NUM_WORKERS = NUM_CORES * NUM_SUBCORES
K = 16


def _sc_transe(total):
    per_w = total // NUM_WORKERS
    n_batches = per_w // K

    mesh = plsc.VectorSubcoreMesh(
        core_axis_name="c", subcore_axis_name="s",
        num_cores=NUM_CORES, num_subcores=NUM_SUBCORES)

    @functools.partial(
        pl.kernel,
        out_type=jax.ShapeDtypeStruct((total,), jnp.float32),
        mesh=mesh,
        compiler_params=pltpu.CompilerParams(needs_layout_passes=False),
        scratch_types=[
            pltpu.VMEM((per_w,), jnp.int32),
            pltpu.VMEM((per_w,), jnp.int32),
            pltpu.VMEM((per_w,), jnp.int32),
            pltpu.VMEM((K, DIM), jnp.float32),
            pltpu.VMEM((K, DIM), jnp.float32),
            pltpu.VMEM((K, DIM), jnp.float32),
            pltpu.VMEM((per_w,), jnp.float32),
            pltpu.SemaphoreType.DMA,
        ],
    )
    def k(hidx_hbm, ridx_hbm, tidx_hbm, ent_hbm, rel_hbm, out_hbm,
          hidx_v, ridx_v, tidx_v, h_v, r_v, t_v, out_v, sem):
        wid = lax.axis_index("s") * NUM_CORES + lax.axis_index("c")
        base = wid * per_w
        pltpu.sync_copy(hidx_hbm.at[pl.ds(base, per_w)], hidx_v)
        pltpu.sync_copy(ridx_hbm.at[pl.ds(base, per_w)], ridx_v)
        pltpu.sync_copy(tidx_hbm.at[pl.ds(base, per_w)], tidx_v)

        lane = jnp.arange(LANES, dtype=jnp.int32)

        def batch_body(b, _):
            b0 = b * K
            hvec = hidx_v[pl.ds(b0, K)]
            rvec = ridx_v[pl.ds(b0, K)]
            tvec = tidx_v[pl.ds(b0, K)]
            for slot in range(K):
                eh = hvec[slot]
                er = rvec[slot]
                et = tvec[slot]
                pltpu.async_copy(
                    ent_hbm.at[pl.ds(eh, 1)], h_v.at[pl.ds(slot, 1)], sem)
                pltpu.async_copy(
                    rel_hbm.at[pl.ds(er, 1)], r_v.at[pl.ds(slot, 1)], sem)
                pltpu.async_copy(
                    ent_hbm.at[pl.ds(et, 1)], t_v.at[pl.ds(slot, 1)], sem)
            pltpu.make_async_copy(ent_hbm.at[pl.ds(0, K)], h_v, sem).wait()
            pltpu.make_async_copy(rel_hbm.at[pl.ds(0, K)], r_v, sem).wait()
            pltpu.make_async_copy(ent_hbm.at[pl.ds(0, K)], t_v, sem).wait()

            def d_body(d, acc):
                col = jnp.full((LANES,), d, dtype=jnp.int32)
                hv = plsc.load_gather(h_v, [lane, col])
                rv = plsc.load_gather(r_v, [lane, col])
                tv = plsc.load_gather(t_v, [lane, col])
                return acc + jnp.abs(hv + rv - tv)

            acc = lax.fori_loop(
                0, DIM, d_body, jnp.zeros((LANES,), jnp.float32))
            out_v[pl.ds(b0, LANES)] = acc
            return 0

        lax.fori_loop(0, n_batches, batch_body, 0)

        pltpu.sync_copy(out_v, out_hbm.at[pl.ds(base, per_w)])

    return k


def kernel(positive_triplets, negative_triplets, entities_emb, relations_emb):
    batch = positive_triplets.shape[0]
    total = 2 * batch
    trip = jnp.concatenate(
        [positive_triplets, negative_triplets], axis=0).astype(jnp.int32)

    out = _sc_transe(total)(
        trip[:, 0], trip[:, 1], trip[:, 2], entities_emb, relations_emb)
    return out[:batch], out[batch:]


# speedup vs baseline: 1.5006x; 1.0556x over previous
import functools

import jax
import jax.numpy as jnp
from jax import lax
from jax.experimental import pallas as pl
from jax.experimental.pallas import tpu as pltpu
from jax.experimental.pallas import tpu_sc as plsc

DIM = 64
LANES = 16
NUM_CORES = 2
NUM_SUBCORES = 16
NUM_WORKERS = NUM_CORES * NUM_SUBCORES
K = 32


def _sc_transe(total):
    per_w = total // NUM_WORKERS
    n_batches = per_w // K
    assert n_batches % 2 == 0

    mesh = plsc.VectorSubcoreMesh(
        core_axis_name="c", subcore_axis_name="s",
        num_cores=NUM_CORES, num_subcores=NUM_SUBCORES)

    @functools.partial(
        pl.kernel,
        out_type=jax.ShapeDtypeStruct((total,), jnp.float32),
        mesh=mesh,
        compiler_params=pltpu.CompilerParams(needs_layout_passes=False),
        scratch_types=[
            pltpu.VMEM((per_w,), jnp.int32),
            pltpu.VMEM((per_w,), jnp.int32),
            pltpu.VMEM((per_w,), jnp.int32),
            pltpu.VMEM((K, DIM), jnp.float32),
            pltpu.VMEM((K, DIM), jnp.float32),
            pltpu.VMEM((K, DIM), jnp.float32),
            pltpu.VMEM((K, DIM), jnp.float32),
            pltpu.VMEM((K, DIM), jnp.float32),
            pltpu.VMEM((K, DIM), jnp.float32),
            pltpu.VMEM((per_w,), jnp.float32),
            pltpu.SemaphoreType.DMA,
            pltpu.SemaphoreType.DMA,
        ],
    )
    def k(hidx_hbm, ridx_hbm, tidx_hbm, ent_hbm, rel_hbm, out_hbm,
          hidx_v, ridx_v, tidx_v, h0, r0, t0, h1, r1, t1, out_v,
          sem0, sem1):
        wid = lax.axis_index("s") * NUM_CORES + lax.axis_index("c")
        base = wid * per_w
        pltpu.sync_copy(hidx_hbm.at[pl.ds(base, per_w)], hidx_v)
        pltpu.sync_copy(ridx_hbm.at[pl.ds(base, per_w)], ridx_v)
        pltpu.sync_copy(tidx_hbm.at[pl.ds(base, per_w)], tidx_v)

        lane = jnp.arange(LANES, dtype=jnp.int32)
        bufs = ((h0, r0, t0, sem0), (h1, r1, t1, sem1))

        def issue(b, buf):
            h_b, r_b, t_b, sem = buf
            b0 = b * K
            for g in range(K // LANES):
                hvec = hidx_v[pl.ds(b0 + g * LANES, LANES)]
                rvec = ridx_v[pl.ds(b0 + g * LANES, LANES)]
                tvec = tidx_v[pl.ds(b0 + g * LANES, LANES)]
                for i in range(LANES):
                    slot = g * LANES + i
                    pltpu.async_copy(ent_hbm.at[pl.ds(hvec[i], 1)],
                                     h_b.at[pl.ds(slot, 1)], sem)
                    pltpu.async_copy(rel_hbm.at[pl.ds(rvec[i], 1)],
                                     r_b.at[pl.ds(slot, 1)], sem)
                    pltpu.async_copy(ent_hbm.at[pl.ds(tvec[i], 1)],
                                     t_b.at[pl.ds(slot, 1)], sem)

        def drain_compute(b, buf):
            h_b, r_b, t_b, sem = buf
            pltpu.make_async_copy(ent_hbm.at[pl.ds(0, K)], h_b, sem).wait()
            pltpu.make_async_copy(rel_hbm.at[pl.ds(0, K)], r_b, sem).wait()
            pltpu.make_async_copy(ent_hbm.at[pl.ds(0, K)], t_b, sem).wait()
            for g in range(K // LANES):
                rows = g * LANES + lane

                def d_body(d, acc, rows=rows):
                    col = jnp.full((LANES,), d, dtype=jnp.int32)
                    hv = plsc.load_gather(h_b, [rows, col])
                    rv = plsc.load_gather(r_b, [rows, col])
                    tv = plsc.load_gather(t_b, [rows, col])
                    return acc + jnp.abs(hv + rv - tv)

                acc = lax.fori_loop(
                    0, DIM, d_body, jnp.zeros((LANES,), jnp.float32))
                out_v[pl.ds(b * K + g * LANES, LANES)] = acc

        issue(0, bufs[0])

        def pair_body(p, _):
            b = p * 2
            issue(b + 1, bufs[1])
            drain_compute(b, bufs[0])

            @pl.when(b + 2 < n_batches)
            def _():
                issue(b + 2, bufs[0])

            drain_compute(b + 1, bufs[1])
            return 0

        lax.fori_loop(0, n_batches // 2, pair_body, 0)

        pltpu.sync_copy(out_v, out_hbm.at[pl.ds(base, per_w)])

    return k


def kernel(positive_triplets, negative_triplets, entities_emb, relations_emb):
    batch = positive_triplets.shape[0]
    total = 2 * batch
    trip = jnp.concatenate(
        [positive_triplets, negative_triplets], axis=0).astype(jnp.int32)

    out = _sc_transe(total)(
        trip[:, 0], trip[:, 1], trip[:, 2], entities_emb, relations_emb)
    return out[:batch], out[batch:]


# speedup vs baseline: 1.5034x; 1.0018x over previous
import functools

import jax
import jax.numpy as jnp
from jax import lax
from jax.experimental import pallas as pl
from jax.experimental.pallas import tpu as pltpu
from jax.experimental.pallas import tpu_sc as plsc

DIM = 64
LANES = 16
NUM_CORES = 2
NUM_SUBCORES = 16
NUM_WORKERS = NUM_CORES * NUM_SUBCORES
K = 32
NSEM = 6


def _sc_transe(total):
    per_w = total // NUM_WORKERS
    n_batches = per_w // K
    assert n_batches % 2 == 0
    assert (3 * K) % NSEM == 0

    mesh = plsc.VectorSubcoreMesh(
        core_axis_name="c", subcore_axis_name="s",
        num_cores=NUM_CORES, num_subcores=NUM_SUBCORES)

    @functools.partial(
        pl.kernel,
        out_type=jax.ShapeDtypeStruct((total,), jnp.float32),
        mesh=mesh,
        compiler_params=pltpu.CompilerParams(needs_layout_passes=False),
        scratch_types=[
            pltpu.VMEM((per_w,), jnp.int32),
            pltpu.VMEM((per_w,), jnp.int32),
            pltpu.VMEM((per_w,), jnp.int32),
            pltpu.VMEM((K, DIM), jnp.float32),
            pltpu.VMEM((K, DIM), jnp.float32),
            pltpu.VMEM((K, DIM), jnp.float32),
            pltpu.VMEM((K, DIM), jnp.float32),
            pltpu.VMEM((K, DIM), jnp.float32),
            pltpu.VMEM((K, DIM), jnp.float32),
            pltpu.VMEM((per_w,), jnp.float32),
            [pltpu.SemaphoreType.DMA] * NSEM,
            [pltpu.SemaphoreType.DMA] * NSEM,
        ],
    )
    def k(hidx_hbm, ridx_hbm, tidx_hbm, ent_hbm, rel_hbm, out_hbm,
          hidx_v, ridx_v, tidx_v, h0, r0, t0, h1, r1, t1, out_v,
          sems0, sems1):
        wid = lax.axis_index("s") * NUM_CORES + lax.axis_index("c")
        base = wid * per_w
        pltpu.sync_copy(hidx_hbm.at[pl.ds(base, per_w)], hidx_v)
        pltpu.sync_copy(ridx_hbm.at[pl.ds(base, per_w)], ridx_v)
        pltpu.sync_copy(tidx_hbm.at[pl.ds(base, per_w)], tidx_v)

        lane = jnp.arange(LANES, dtype=jnp.int32)
        bufs = ((h0, r0, t0, sems0), (h1, r1, t1, sems1))

        def issue(b, buf):
            h_b, r_b, t_b, sems = buf
            b0 = b * K
            n = 0
            for g in range(K // LANES):
                hvec = hidx_v[pl.ds(b0 + g * LANES, LANES)]
                rvec = ridx_v[pl.ds(b0 + g * LANES, LANES)]
                tvec = tidx_v[pl.ds(b0 + g * LANES, LANES)]
                for i in range(LANES):
                    slot = g * LANES + i
                    pltpu.async_copy(ent_hbm.at[pl.ds(hvec[i], 1)],
                                     h_b.at[pl.ds(slot, 1)], sems[n % NSEM])
                    n += 1
                    pltpu.async_copy(rel_hbm.at[pl.ds(rvec[i], 1)],
                                     r_b.at[pl.ds(slot, 1)], sems[n % NSEM])
                    n += 1
                    pltpu.async_copy(ent_hbm.at[pl.ds(tvec[i], 1)],
                                     t_b.at[pl.ds(slot, 1)], sems[n % NSEM])
                    n += 1

        def drain_compute(b, buf):
            h_b, r_b, t_b, sems = buf
            rows_per_sem = (3 * K) // NSEM
            for s in range(NSEM):
                pltpu.make_async_copy(
                    ent_hbm.at[pl.ds(0, rows_per_sem)],
                    h_b.at[pl.ds(0, rows_per_sem)], sems[s]).wait()
            for g in range(K // LANES):
                rows = g * LANES + lane

                def d_body(d, acc, rows=rows):
                    col = jnp.full((LANES,), d, dtype=jnp.int32)
                    hv = plsc.load_gather(h_b, [rows, col])
                    rv = plsc.load_gather(r_b, [rows, col])
                    tv = plsc.load_gather(t_b, [rows, col])
                    return acc + jnp.abs(hv + rv - tv)

                acc = lax.fori_loop(
                    0, DIM, d_body, jnp.zeros((LANES,), jnp.float32))
                out_v[pl.ds(b * K + g * LANES, LANES)] = acc

        issue(0, bufs[0])

        def pair_body(p, _):
            b = p * 2
            issue(b + 1, bufs[1])
            drain_compute(b, bufs[0])

            @pl.when(b + 2 < n_batches)
            def _():
                issue(b + 2, bufs[0])

            drain_compute(b + 1, bufs[1])
            return 0

        lax.fori_loop(0, n_batches // 2, pair_body, 0)

        pltpu.sync_copy(out_v, out_hbm.at[pl.ds(base, per_w)])

    return k


def kernel(positive_triplets, negative_triplets, entities_emb, relations_emb):
    batch = positive_triplets.shape[0]
    total = 2 * batch
    trip = jnp.concatenate(
        [positive_triplets, negative_triplets], axis=0).astype(jnp.int32)

    out = _sc_transe(total)(
        trip[:, 0], trip[:, 1], trip[:, 2], entities_emb, relations_emb)
    return out[:batch], out[batch:]
